# Initial kernel scaffold; baseline (speedup 1.0000x reference)
import jax
import jax.numpy as jnp
from jax.experimental import pallas as pl


def kernel(pos, edge_index, cell, cell_shift, batch):
    raise NotImplementedError("write your pallas kernel here")


# speedup vs baseline: 11.1592x; 11.1592x over previous
import functools

import jax
import jax.numpy as jnp
from jax import lax
from jax.experimental import pallas as pl
from jax.experimental.pallas import tpu as pltpu
from jax.experimental.pallas import tpu_sc as plsc

N = 100000
E = 6400000
NCELL = 16

NC = 2
NS = 16
NW = NC * NS
PER_W = E // NW
K = 2000
NCHUNK = PER_W // K
SUB = 80
NSUB = K // SUB
GPS = SUB // 16

_mesh = plsc.VectorSubcoreMesh(core_axis_name="c", subcore_axis_name="s")


def _c16(v):
    return jnp.full((16,), v, jnp.int32)


@functools.partial(
    pl.kernel,
    out_type=(
        jax.ShapeDtypeStruct((3 * E,), jnp.float32),
        jax.ShapeDtypeStruct((E,), jnp.float32),
    ),
    mesh=_mesh,
    compiler_params=pltpu.CompilerParams(
        needs_layout_passes=False, use_tc_tiling_on_sc=False),
    scratch_types=[
        pltpu.VMEM((NSUB, SUB), jnp.int32),
        pltpu.VMEM((NSUB, SUB), jnp.int32),
        pltpu.VMEM((NSUB, SUB, 8), jnp.float32),
        pltpu.VMEM((NSUB, SUB, 8), jnp.float32),
        pltpu.VMEM((3 * K,), jnp.float32),
        pltpu.VMEM((NCELL * 9,), jnp.float32),
        pltpu.VMEM((3 * K,), jnp.float32),
        pltpu.VMEM((K,), jnp.float32),
        pltpu.SemaphoreType.DMA,
    ],
)
def _edge_kernel(pos4, idxsrc, idxdst, cellflat, cellshift,
                 vec_out, len_out,
                 idxs_v, idxd_v, srcr, dstr, shf, cellv, vecv, lenv, sem):
    wid = lax.axis_index("s") * NC + lax.axis_index("c")
    lanes = lax.iota(jnp.int32, 16)

    pltpu.sync_copy(cellflat, cellv)

    def chunk_body(c, carry):
        base = wid * PER_W + c * K
        rowbase = base // SUB
        pltpu.sync_copy(idxsrc.at[pl.ds(rowbase, NSUB)], idxs_v)
        pltpu.sync_copy(idxdst.at[pl.ds(rowbase, NSUB)], idxd_v)
        pltpu.sync_copy(cellshift.at[pl.ds(3 * base, 3 * K)], shf)

        def gather(j, cr):
            a = pltpu.async_copy(pos4.at[idxs_v.at[j]], srcr.at[j], sem)
            b2 = pltpu.async_copy(pos4.at[idxd_v.at[j]], dstr.at[j], sem)
            a.wait()
            b2.wait()
            return cr

        lax.fori_loop(0, NSUB, gather, 0)

        def group(g, cr):
            sub = g // GPS
            subv = jnp.full((16,), sub, jnp.int32)
            rowv = (g % GPS) * 16 + lanes
            ridx = g * 16 + lanes
            sx = plsc.load_gather(srcr, [subv, rowv, _c16(0)])
            sy = plsc.load_gather(srcr, [subv, rowv, _c16(1)])
            sz = plsc.load_gather(srcr, [subv, rowv, _c16(2)])
            b = plsc.load_gather(srcr, [subv, rowv, _c16(3)]).astype(jnp.int32)
            dx = plsc.load_gather(dstr, [subv, rowv, _c16(0)])
            dy = plsc.load_gather(dstr, [subv, rowv, _c16(1)])
            dz = plsc.load_gather(dstr, [subv, rowv, _c16(2)])
            r3 = ridx * 3
            s0 = plsc.load_gather(shf, [r3])
            s1 = plsc.load_gather(shf, [r3 + 1])
            s2 = plsc.load_gather(shf, [r3 + 2])
            cb = b * 9
            vx = dx - sx \
                + s0 * plsc.load_gather(cellv, [cb]) \
                + s1 * plsc.load_gather(cellv, [cb + 3]) \
                + s2 * plsc.load_gather(cellv, [cb + 6])
            vy = dy - sy \
                + s0 * plsc.load_gather(cellv, [cb + 1]) \
                + s1 * plsc.load_gather(cellv, [cb + 4]) \
                + s2 * plsc.load_gather(cellv, [cb + 7])
            vz = dz - sz \
                + s0 * plsc.load_gather(cellv, [cb + 2]) \
                + s1 * plsc.load_gather(cellv, [cb + 5]) \
                + s2 * plsc.load_gather(cellv, [cb + 8])
            nsq = vx * vx + vy * vy + vz * vz
            yi = jnp.int32(0x5F3759DF) - (plsc.bitcast(nsq, jnp.int32) >> 1)
            y = plsc.bitcast(yi, jnp.float32)
            y = y * (1.5 - 0.5 * nsq * y * y)
            y = y * (1.5 - 0.5 * nsq * y * y)
            y = y * (1.5 - 0.5 * nsq * y * y)
            ln = jnp.where(nsq > 0.0, nsq * y, 0.0)
            plsc.store_scatter(vecv, [r3], vx)
            plsc.store_scatter(vecv, [r3 + 1], vy)
            plsc.store_scatter(vecv, [r3 + 2], vz)
            plsc.store_scatter(lenv, [ridx], ln)
            return cr

        lax.fori_loop(0, K // 16, group, 0)

        pltpu.sync_copy(vecv, vec_out.at[pl.ds(3 * base, 3 * K)])
        pltpu.sync_copy(lenv, len_out.at[pl.ds(base, K)])
        return carry

    lax.fori_loop(0, NCHUNK, chunk_body, 0)


def kernel(pos, edge_index, cell, cell_shift, batch):
    pos4 = jnp.concatenate(
        [pos, batch.astype(jnp.float32)[:, None],
         jnp.zeros((pos.shape[0], 4), jnp.float32)], axis=1)
    cellflat = cell.reshape(-1)
    idxsrc = edge_index[0].reshape(E // SUB, SUB)
    idxdst = edge_index[1].reshape(E // SUB, SUB)
    vecflat, length = _edge_kernel(pos4, idxsrc, idxdst,
                                   cellflat, cell_shift.reshape(-1))
    return vecflat.reshape(-1, 3), length


# speedup vs baseline: 12.8223x; 1.1490x over previous
import functools

import jax
import jax.numpy as jnp
from jax import lax
from jax.experimental import pallas as pl
from jax.experimental.pallas import tpu as pltpu
from jax.experimental.pallas import tpu_sc as plsc

N = 100000
E = 6400000
NCELL = 16

NC = 2
NS = 16
NW = NC * NS
PER_W = E // NW
K = 2000
NCHUNK = PER_W // K
SUB = 80
NSUB = K // SUB
GPS = SUB // 16

_mesh = plsc.VectorSubcoreMesh(core_axis_name="c", subcore_axis_name="s")


def _c16(v):
    return jnp.full((16,), v, jnp.int32)


_BUF = [
    pltpu.VMEM((NSUB, SUB), jnp.int32),
    pltpu.VMEM((NSUB, SUB), jnp.int32),
    pltpu.VMEM((NSUB, SUB, 8), jnp.float32),
    pltpu.VMEM((NSUB, SUB, 8), jnp.float32),
    pltpu.VMEM((3 * K,), jnp.float32),
    pltpu.VMEM((3 * K,), jnp.float32),
    pltpu.VMEM((K,), jnp.float32),
    pltpu.SemaphoreType.DMA,
    pltpu.SemaphoreType.DMA,
    pltpu.SemaphoreType.DMA,
]


@functools.partial(
    pl.kernel,
    out_type=(
        jax.ShapeDtypeStruct((3 * E,), jnp.float32),
        jax.ShapeDtypeStruct((E,), jnp.float32),
    ),
    mesh=_mesh,
    compiler_params=pltpu.CompilerParams(
        needs_layout_passes=False, use_tc_tiling_on_sc=False),
    scratch_types=_BUF + _BUF + [pltpu.VMEM((NCELL * 9,), jnp.float32)],
)
def _edge_kernel(pos4, idxsrc, idxdst, cellflat, cellshift,
                 vec_out, len_out, *scratch):
    buf0 = tuple(scratch[:10])
    buf1 = tuple(scratch[10:20])
    cellv = scratch[20]
    wid = lax.axis_index("s") * NC + lax.axis_index("c")
    lanes = lax.iota(jnp.int32, 16)

    pltpu.sync_copy(cellflat, cellv)

    def base_of(c):
        return wid * PER_W + c * K

    def start_in(c, B):
        idxs_v, idxd_v, _, _, shf = B[:5]
        sem_in = B[7]
        base = base_of(c)
        rowbase = base // SUB
        pltpu.async_copy(idxsrc.at[pl.ds(rowbase, NSUB)], idxs_v, sem_in)
        pltpu.async_copy(idxdst.at[pl.ds(rowbase, NSUB)], idxd_v, sem_in)
        pltpu.async_copy(cellshift.at[pl.ds(3 * base, 3 * K)], shf, sem_in)

    def wait_in(B):
        idxs_v, idxd_v, _, _, shf = B[:5]
        sem_in = B[7]
        pltpu.make_async_copy(idxsrc.at[pl.ds(0, NSUB)], idxs_v, sem_in).wait()
        pltpu.make_async_copy(idxdst.at[pl.ds(0, NSUB)], idxd_v, sem_in).wait()
        pltpu.make_async_copy(
            cellshift.at[pl.ds(0, 3 * K)], shf, sem_in).wait()

    def fire_g(B):
        idxs_v, idxd_v, srcr, dstr = B[:4]
        sem_g = B[8]

        def fire(j, cr):
            pltpu.async_copy(pos4.at[idxs_v.at[j]], srcr.at[j], sem_g)
            pltpu.async_copy(pos4.at[idxd_v.at[j]], dstr.at[j], sem_g)
            return cr

        lax.fori_loop(0, NSUB, fire, 0)

    def drain_g(B):
        idxs_v, idxd_v, srcr, dstr = B[:4]
        sem_g = B[8]

        def drain(j, cr):
            pltpu.make_async_copy(
                pos4.at[idxs_v.at[j]], srcr.at[j], sem_g).wait()
            pltpu.make_async_copy(
                pos4.at[idxd_v.at[j]], dstr.at[j], sem_g).wait()
            return cr

        lax.fori_loop(0, NSUB, drain, 0)

    def compute(B):
        srcr, dstr = B[2], B[3]
        shf, vecv, lenv = B[4], B[5], B[6]

        def group(g, cr):
            sub = g // GPS
            subv = jnp.full((16,), sub, jnp.int32)
            rowv = (g % GPS) * 16 + lanes
            ridx = g * 16 + lanes
            sx = plsc.load_gather(srcr, [subv, rowv, _c16(0)])
            sy = plsc.load_gather(srcr, [subv, rowv, _c16(1)])
            sz = plsc.load_gather(srcr, [subv, rowv, _c16(2)])
            b = plsc.load_gather(
                srcr, [subv, rowv, _c16(3)]).astype(jnp.int32)
            dx = plsc.load_gather(dstr, [subv, rowv, _c16(0)])
            dy = plsc.load_gather(dstr, [subv, rowv, _c16(1)])
            dz = plsc.load_gather(dstr, [subv, rowv, _c16(2)])
            r3 = ridx * 3
            s0 = plsc.load_gather(shf, [r3])
            s1 = plsc.load_gather(shf, [r3 + 1])
            s2 = plsc.load_gather(shf, [r3 + 2])
            cb = b * 9
            vx = dx - sx \
                + s0 * plsc.load_gather(cellv, [cb]) \
                + s1 * plsc.load_gather(cellv, [cb + 3]) \
                + s2 * plsc.load_gather(cellv, [cb + 6])
            vy = dy - sy \
                + s0 * plsc.load_gather(cellv, [cb + 1]) \
                + s1 * plsc.load_gather(cellv, [cb + 4]) \
                + s2 * plsc.load_gather(cellv, [cb + 7])
            vz = dz - sz \
                + s0 * plsc.load_gather(cellv, [cb + 2]) \
                + s1 * plsc.load_gather(cellv, [cb + 5]) \
                + s2 * plsc.load_gather(cellv, [cb + 8])
            nsq = vx * vx + vy * vy + vz * vz
            yi = jnp.int32(0x5F3759DF) - (plsc.bitcast(nsq, jnp.int32) >> 1)
            y = plsc.bitcast(yi, jnp.float32)
            y = y * (1.5 - 0.5 * nsq * y * y)
            y = y * (1.5 - 0.5 * nsq * y * y)
            y = y * (1.5 - 0.5 * nsq * y * y)
            ln = jnp.where(nsq > 0.0, nsq * y, 0.0)
            plsc.store_scatter(vecv, [r3], vx)
            plsc.store_scatter(vecv, [r3 + 1], vy)
            plsc.store_scatter(vecv, [r3 + 2], vz)
            plsc.store_scatter(lenv, [ridx], ln)
            return cr

        lax.fori_loop(0, K // 16, group, 0)

    def start_out(c, B):
        vecv, lenv = B[5], B[6]
        sem_out = B[9]
        base = base_of(c)
        pltpu.async_copy(vecv, vec_out.at[pl.ds(3 * base, 3 * K)], sem_out)
        pltpu.async_copy(lenv, len_out.at[pl.ds(base, K)], sem_out)

    def wait_out(B):
        vecv, lenv = B[5], B[6]
        sem_out = B[9]
        pltpu.make_async_copy(
            vecv, vec_out.at[pl.ds(0, 3 * K)], sem_out).wait()
        pltpu.make_async_copy(lenv, len_out.at[pl.ds(0, K)], sem_out).wait()

    def step(c, B, NB, do_next, do_waitout, do_startin):
        drain_g(B)
        if do_next:
            wait_in(NB)
            fire_g(NB)
        if do_waitout:
            wait_out(B)
        compute(B)
        start_out(c, B)
        if do_startin:
            start_in(c + 2, B)

    start_in(0, buf0)
    start_in(1, buf1)
    wait_in(buf0)
    fire_g(buf0)
    step(0, buf0, buf1, True, False, True)
    step(1, buf1, buf0, True, False, True)

    def body(c2, carry):
        c = 2 * c2
        step(c, buf0, buf1, True, True, True)
        step(c + 1, buf1, buf0, True, True, True)
        return carry

    lax.fori_loop(1, NCHUNK // 2 - 1, body, 0)
    step(NCHUNK - 2, buf0, buf1, True, True, False)
    step(NCHUNK - 1, buf1, buf0, False, True, False)
    wait_out(buf0)
    wait_out(buf1)


def kernel(pos, edge_index, cell, cell_shift, batch):
    pos4 = jnp.concatenate(
        [pos, batch.astype(jnp.float32)[:, None],
         jnp.zeros((pos.shape[0], 4), jnp.float32)], axis=1)
    cellflat = cell.reshape(-1)
    idxsrc = edge_index[0].reshape(E // SUB, SUB)
    idxdst = edge_index[1].reshape(E // SUB, SUB)
    vecflat, length = _edge_kernel(pos4, idxsrc, idxdst,
                                   cellflat, cell_shift.reshape(-1))
    return vecflat.reshape(-1, 3), length


# speedup vs baseline: 42.0545x; 3.2798x over previous
import functools

import jax
import jax.numpy as jnp
from jax import lax
from jax.experimental import pallas as pl
from jax.experimental.pallas import tpu as pltpu
from jax.experimental.pallas import tpu_sc as plsc

N = 100000
E = 6400000
NCELL = 16

NC = 2
NS = 16
NW = NC * NS
PER_W = E // NW
K = 2000
NCHUNK = PER_W // K
SUB = 80
NSUB = K // SUB
GPS = SUB // 16

_mesh = plsc.VectorSubcoreMesh(core_axis_name="c", subcore_axis_name="s")


def _c16(v):
    return jnp.full((16,), v, jnp.int32)


_BUF = [
    pltpu.VMEM((NSUB, SUB), jnp.int32),
    pltpu.VMEM((NSUB, SUB), jnp.int32),
    pltpu.VMEM((NSUB, SUB, 8), jnp.float32),
    pltpu.VMEM((NSUB, SUB, 8), jnp.float32),
    pltpu.VMEM((3, K), jnp.float32),
    pltpu.VMEM((3, K), jnp.float32),
    pltpu.VMEM((K,), jnp.float32),
    pltpu.SemaphoreType.DMA,
    pltpu.SemaphoreType.DMA,
    pltpu.SemaphoreType.DMA,
]


@functools.partial(
    pl.kernel,
    out_type=(
        jax.ShapeDtypeStruct((3, E), jnp.float32),
        jax.ShapeDtypeStruct((E,), jnp.float32),
    ),
    mesh=_mesh,
    compiler_params=pltpu.CompilerParams(
        needs_layout_passes=False, use_tc_tiling_on_sc=False),
    scratch_types=_BUF + _BUF + [pltpu.VMEM((NCELL * 9,), jnp.float32)],
)
def _edge_kernel(pos4, edge3, cellflat, cst,
                 vec_out, len_out, *scratch):
    buf0 = tuple(scratch[:10])
    buf1 = tuple(scratch[10:20])
    cellv = scratch[20]
    wid = lax.axis_index("s") * NC + lax.axis_index("c")
    lanes = lax.iota(jnp.int32, 16)

    pltpu.sync_copy(cellflat, cellv)

    def base_of(c):
        return wid * PER_W + c * K

    def start_in(c, B):
        idxs_v, idxd_v, _, _, shf = B[:5]
        sem_in = B[7]
        base = base_of(c)
        rowbase = base // SUB
        pltpu.async_copy(edge3.at[0, pl.ds(rowbase, NSUB)], idxs_v, sem_in)
        pltpu.async_copy(edge3.at[1, pl.ds(rowbase, NSUB)], idxd_v, sem_in)
        for i in range(3):
            pltpu.async_copy(cst.at[i, pl.ds(base, K)], shf.at[i], sem_in)

    def wait_in(B):
        idxs_v, idxd_v, _, _, shf = B[:5]
        sem_in = B[7]
        pltpu.make_async_copy(
            edge3.at[0, pl.ds(0, NSUB)], idxs_v, sem_in).wait()
        pltpu.make_async_copy(
            edge3.at[1, pl.ds(0, NSUB)], idxd_v, sem_in).wait()
        for i in range(3):
            pltpu.make_async_copy(
                cst.at[i, pl.ds(0, K)], shf.at[i], sem_in).wait()

    def fire_g(B):
        idxs_v, idxd_v, srcr, dstr = B[:4]
        sem_g = B[8]

        def fire(j, cr):
            pltpu.async_copy(pos4.at[idxs_v.at[j]], srcr.at[j], sem_g)
            pltpu.async_copy(pos4.at[idxd_v.at[j]], dstr.at[j], sem_g)
            return cr

        lax.fori_loop(0, NSUB, fire, 0)

    def drain_g(B):
        idxs_v, idxd_v, srcr, dstr = B[:4]
        sem_g = B[8]

        def drain(j, cr):
            pltpu.make_async_copy(
                pos4.at[idxs_v.at[j]], srcr.at[j], sem_g).wait()
            pltpu.make_async_copy(
                pos4.at[idxd_v.at[j]], dstr.at[j], sem_g).wait()
            return cr

        lax.fori_loop(0, NSUB, drain, 0)

    def compute(B):
        srcr, dstr = B[2], B[3]
        shf, vecv, lenv = B[4], B[5], B[6]

        def group(g, cr):
            sub = g // GPS
            subv = jnp.full((16,), sub, jnp.int32)
            rowv = (g % GPS) * 16 + lanes
            ridx = g * 16 + lanes
            sx = plsc.load_gather(srcr, [subv, rowv, _c16(0)])
            sy = plsc.load_gather(srcr, [subv, rowv, _c16(1)])
            sz = plsc.load_gather(srcr, [subv, rowv, _c16(2)])
            b = plsc.load_gather(
                srcr, [subv, rowv, _c16(3)]).astype(jnp.int32)
            dx = plsc.load_gather(dstr, [subv, rowv, _c16(0)])
            dy = plsc.load_gather(dstr, [subv, rowv, _c16(1)])
            dz = plsc.load_gather(dstr, [subv, rowv, _c16(2)])
            s0 = plsc.load_gather(shf, [_c16(0), ridx])
            s1 = plsc.load_gather(shf, [_c16(1), ridx])
            s2 = plsc.load_gather(shf, [_c16(2), ridx])
            cb = b * 9
            vx = dx - sx \
                + s0 * plsc.load_gather(cellv, [cb]) \
                + s1 * plsc.load_gather(cellv, [cb + 3]) \
                + s2 * plsc.load_gather(cellv, [cb + 6])
            vy = dy - sy \
                + s0 * plsc.load_gather(cellv, [cb + 1]) \
                + s1 * plsc.load_gather(cellv, [cb + 4]) \
                + s2 * plsc.load_gather(cellv, [cb + 7])
            vz = dz - sz \
                + s0 * plsc.load_gather(cellv, [cb + 2]) \
                + s1 * plsc.load_gather(cellv, [cb + 5]) \
                + s2 * plsc.load_gather(cellv, [cb + 8])
            nsq = vx * vx + vy * vy + vz * vz
            yi = jnp.int32(0x5F3759DF) - (plsc.bitcast(nsq, jnp.int32) >> 1)
            y = plsc.bitcast(yi, jnp.float32)
            y = y * (1.5 - 0.5 * nsq * y * y)
            y = y * (1.5 - 0.5 * nsq * y * y)
            y = y * (1.5 - 0.5 * nsq * y * y)
            ln = jnp.where(nsq > 0.0, nsq * y, 0.0)
            plsc.store_scatter(vecv, [_c16(0), ridx], vx)
            plsc.store_scatter(vecv, [_c16(1), ridx], vy)
            plsc.store_scatter(vecv, [_c16(2), ridx], vz)
            plsc.store_scatter(lenv, [ridx], ln)
            return cr

        lax.fori_loop(0, K // 16, group, 0)

    def start_out(c, B):
        vecv, lenv = B[5], B[6]
        sem_out = B[9]
        base = base_of(c)
        for i in range(3):
            pltpu.async_copy(
                vecv.at[i], vec_out.at[i, pl.ds(base, K)], sem_out)
        pltpu.async_copy(lenv, len_out.at[pl.ds(base, K)], sem_out)

    def wait_out(B):
        vecv, lenv = B[5], B[6]
        sem_out = B[9]
        for i in range(3):
            pltpu.make_async_copy(
                vecv.at[i], vec_out.at[i, pl.ds(0, K)], sem_out).wait()
        pltpu.make_async_copy(lenv, len_out.at[pl.ds(0, K)], sem_out).wait()

    def step(c, B, NB, do_next, do_waitout, do_startin):
        drain_g(B)
        if do_next:
            wait_in(NB)
            fire_g(NB)
        if do_waitout:
            wait_out(B)
        compute(B)
        start_out(c, B)
        if do_startin:
            start_in(c + 2, B)

    start_in(0, buf0)
    start_in(1, buf1)
    wait_in(buf0)
    fire_g(buf0)
    step(0, buf0, buf1, True, False, True)
    step(1, buf1, buf0, True, False, True)

    def body(c2, carry):
        c = 2 * c2
        step(c, buf0, buf1, True, True, True)
        step(c + 1, buf1, buf0, True, True, True)
        return carry

    lax.fori_loop(1, NCHUNK // 2 - 1, body, 0)
    step(NCHUNK - 2, buf0, buf1, True, True, False)
    step(NCHUNK - 1, buf1, buf0, False, True, False)
    wait_out(buf0)
    wait_out(buf1)


def kernel(pos, edge_index, cell, cell_shift, batch):
    pos4 = jnp.concatenate(
        [pos, batch.astype(jnp.float32)[:, None],
         jnp.zeros((pos.shape[0], 4), jnp.float32)], axis=1)
    cellflat = cell.reshape(-1)
    edge3 = edge_index.reshape(2, E // SUB, SUB)
    vec3, length = _edge_kernel(pos4, edge3, cellflat, cell_shift.T)
    return vec3.T, length


# speedup vs baseline: 152.2760x; 3.6209x over previous
import functools

import jax
import jax.numpy as jnp
from jax import lax
from jax.experimental import pallas as pl
from jax.experimental.pallas import tpu as pltpu
from jax.experimental.pallas import tpu_sc as plsc

N = 100000
E = 6400000
NCELL = 16

NC = 2
NS = 16
NW = NC * NS
PER_W = E // NW
K = 2000
NCHUNK = PER_W // K
SUB = 80
NSUB = K // SUB
GPS = SUB // 16

_mesh = plsc.VectorSubcoreMesh(core_axis_name="c", subcore_axis_name="s")


def _c16(v):
    return jnp.full((16,), v, jnp.int32)


_BUF = [
    pltpu.VMEM((NSUB, SUB), jnp.int32),
    pltpu.VMEM((NSUB, SUB), jnp.int32),
    pltpu.VMEM((NSUB, SUB, 8), jnp.float32),
    pltpu.VMEM((NSUB, SUB, 8), jnp.float32),
    pltpu.VMEM((3, K), jnp.float32),
    pltpu.VMEM((3, K), jnp.float32),
    pltpu.VMEM((K,), jnp.float32),
    pltpu.SemaphoreType.DMA,
    pltpu.SemaphoreType.DMA,
    pltpu.SemaphoreType.DMA,
]


@functools.partial(
    pl.kernel,
    out_type=(
        jax.ShapeDtypeStruct((E,), jnp.float32),
        jax.ShapeDtypeStruct((E,), jnp.float32),
        jax.ShapeDtypeStruct((E,), jnp.float32),
        jax.ShapeDtypeStruct((E,), jnp.float32),
    ),
    mesh=_mesh,
    compiler_params=pltpu.CompilerParams(
        needs_layout_passes=False, use_tc_tiling_on_sc=False),
    scratch_types=_BUF + _BUF + [pltpu.VMEM((NCELL * 9,), jnp.float32)],
)
def _edge_kernel(pos4, edge3, cellflat, csx, csy, csz,
                 vx_out, vy_out, vz_out, len_out, *scratch):
    buf0 = tuple(scratch[:10])
    buf1 = tuple(scratch[10:20])
    cellv = scratch[20]
    wid = lax.axis_index("s") * NC + lax.axis_index("c")
    lanes = lax.iota(jnp.int32, 16)

    pltpu.sync_copy(cellflat, cellv)

    def base_of(c):
        return wid * PER_W + c * K

    def start_in(c, B):
        idxs_v, idxd_v, _, _, shf = B[:5]
        sem_in = B[7]
        base = base_of(c)
        rowbase = base // SUB
        pltpu.async_copy(edge3.at[0, pl.ds(rowbase, NSUB)], idxs_v, sem_in)
        pltpu.async_copy(edge3.at[1, pl.ds(rowbase, NSUB)], idxd_v, sem_in)
        for i, cs in enumerate((csx, csy, csz)):
            pltpu.async_copy(cs.at[pl.ds(base, K)], shf.at[i], sem_in)

    def wait_in(B):
        idxs_v, idxd_v, _, _, shf = B[:5]
        sem_in = B[7]
        pltpu.make_async_copy(
            edge3.at[0, pl.ds(0, NSUB)], idxs_v, sem_in).wait()
        pltpu.make_async_copy(
            edge3.at[1, pl.ds(0, NSUB)], idxd_v, sem_in).wait()
        for i, cs in enumerate((csx, csy, csz)):
            pltpu.make_async_copy(
                cs.at[pl.ds(0, K)], shf.at[i], sem_in).wait()

    def fire_g(B):
        idxs_v, idxd_v, srcr, dstr = B[:4]
        sem_g = B[8]

        def fire(j, cr):
            pltpu.async_copy(pos4.at[idxs_v.at[j]], srcr.at[j], sem_g)
            pltpu.async_copy(pos4.at[idxd_v.at[j]], dstr.at[j], sem_g)
            return cr

        lax.fori_loop(0, NSUB, fire, 0)

    def drain_g(B):
        idxs_v, idxd_v, srcr, dstr = B[:4]
        sem_g = B[8]

        def drain(j, cr):
            pltpu.make_async_copy(
                pos4.at[idxs_v.at[j]], srcr.at[j], sem_g).wait()
            pltpu.make_async_copy(
                pos4.at[idxd_v.at[j]], dstr.at[j], sem_g).wait()
            return cr

        lax.fori_loop(0, NSUB, drain, 0)

    def compute(B):
        srcr, dstr = B[2], B[3]
        shf, vecv, lenv = B[4], B[5], B[6]

        def group(g, cr):
            sub = g // GPS
            subv = jnp.full((16,), sub, jnp.int32)
            rowv = (g % GPS) * 16 + lanes
            ridx = g * 16 + lanes
            sx = plsc.load_gather(srcr, [subv, rowv, _c16(0)])
            sy = plsc.load_gather(srcr, [subv, rowv, _c16(1)])
            sz = plsc.load_gather(srcr, [subv, rowv, _c16(2)])
            b = plsc.load_gather(
                srcr, [subv, rowv, _c16(3)]).astype(jnp.int32)
            dx = plsc.load_gather(dstr, [subv, rowv, _c16(0)])
            dy = plsc.load_gather(dstr, [subv, rowv, _c16(1)])
            dz = plsc.load_gather(dstr, [subv, rowv, _c16(2)])
            s0 = plsc.load_gather(shf, [_c16(0), ridx])
            s1 = plsc.load_gather(shf, [_c16(1), ridx])
            s2 = plsc.load_gather(shf, [_c16(2), ridx])
            cb = b * 9
            vx = dx - sx \
                + s0 * plsc.load_gather(cellv, [cb]) \
                + s1 * plsc.load_gather(cellv, [cb + 3]) \
                + s2 * plsc.load_gather(cellv, [cb + 6])
            vy = dy - sy \
                + s0 * plsc.load_gather(cellv, [cb + 1]) \
                + s1 * plsc.load_gather(cellv, [cb + 4]) \
                + s2 * plsc.load_gather(cellv, [cb + 7])
            vz = dz - sz \
                + s0 * plsc.load_gather(cellv, [cb + 2]) \
                + s1 * plsc.load_gather(cellv, [cb + 5]) \
                + s2 * plsc.load_gather(cellv, [cb + 8])
            nsq = vx * vx + vy * vy + vz * vz
            yi = jnp.int32(0x5F3759DF) - (plsc.bitcast(nsq, jnp.int32) >> 1)
            y = plsc.bitcast(yi, jnp.float32)
            y = y * (1.5 - 0.5 * nsq * y * y)
            y = y * (1.5 - 0.5 * nsq * y * y)
            y = y * (1.5 - 0.5 * nsq * y * y)
            ln = jnp.where(nsq > 0.0, nsq * y, 0.0)
            plsc.store_scatter(vecv, [_c16(0), ridx], vx)
            plsc.store_scatter(vecv, [_c16(1), ridx], vy)
            plsc.store_scatter(vecv, [_c16(2), ridx], vz)
            plsc.store_scatter(lenv, [ridx], ln)
            return cr

        lax.fori_loop(0, K // 16, group, 0)

    def start_out(c, B):
        vecv, lenv = B[5], B[6]
        sem_out = B[9]
        base = base_of(c)
        for i, vo in enumerate((vx_out, vy_out, vz_out)):
            pltpu.async_copy(vecv.at[i], vo.at[pl.ds(base, K)], sem_out)
        pltpu.async_copy(lenv, len_out.at[pl.ds(base, K)], sem_out)

    def wait_out(B):
        vecv, lenv = B[5], B[6]
        sem_out = B[9]
        for i, vo in enumerate((vx_out, vy_out, vz_out)):
            pltpu.make_async_copy(
                vecv.at[i], vo.at[pl.ds(0, K)], sem_out).wait()
        pltpu.make_async_copy(lenv, len_out.at[pl.ds(0, K)], sem_out).wait()

    def step(c, B, NB, do_next, do_waitout, do_startin):
        drain_g(B)
        if do_next:
            wait_in(NB)
            fire_g(NB)
        if do_waitout:
            wait_out(B)
        compute(B)
        start_out(c, B)
        if do_startin:
            start_in(c + 2, B)

    start_in(0, buf0)
    start_in(1, buf1)
    wait_in(buf0)
    fire_g(buf0)
    step(0, buf0, buf1, True, False, True)
    step(1, buf1, buf0, True, False, True)

    def body(c2, carry):
        c = 2 * c2
        step(c, buf0, buf1, True, True, True)
        step(c + 1, buf1, buf0, True, True, True)
        return carry

    lax.fori_loop(1, NCHUNK // 2 - 1, body, 0)
    step(NCHUNK - 2, buf0, buf1, True, True, False)
    step(NCHUNK - 1, buf1, buf0, False, True, False)
    wait_out(buf0)
    wait_out(buf1)


def kernel(pos, edge_index, cell, cell_shift, batch):
    pos4 = jnp.concatenate(
        [pos, batch.astype(jnp.float32)[:, None],
         jnp.zeros((pos.shape[0], 4), jnp.float32)], axis=1)
    cellflat = cell.reshape(-1)
    edge3 = edge_index.reshape(2, E // SUB, SUB)
    vx, vy, vz, length = _edge_kernel(
        pos4, edge3, cellflat,
        cell_shift[:, 0], cell_shift[:, 1], cell_shift[:, 2])
    return jnp.stack([vx, vy, vz], axis=1), length


# speedup vs baseline: 192.4122x; 1.2636x over previous
import functools

import jax
import jax.numpy as jnp
from jax import lax
from jax.experimental import pallas as pl
from jax.experimental.pallas import tpu as pltpu
from jax.experimental.pallas import tpu_sc as plsc

N = 100000
E = 6400000
NCELL = 16

NC = 2
NS = 16
NW = NC * NS
PER_W = E // NW
K = 2000
NCHUNK = PER_W // K
SUB = 80
NSUB = K // SUB
GPS = SUB // 16

_mesh = plsc.VectorSubcoreMesh(core_axis_name="c", subcore_axis_name="s")


def _c16(v):
    return jnp.full((16,), v, jnp.int32)


_BUF = [
    pltpu.VMEM((NSUB, SUB), jnp.int32),
    pltpu.VMEM((NSUB, SUB), jnp.int32),
    pltpu.VMEM((NSUB, SUB, 8), jnp.float32),
    pltpu.VMEM((NSUB, SUB, 8), jnp.float32),
    pltpu.VMEM((3, K), jnp.float32),
    pltpu.VMEM((3, K), jnp.float32),
    pltpu.VMEM((K,), jnp.float32),
    pltpu.SemaphoreType.DMA,
    pltpu.SemaphoreType.DMA,
    pltpu.SemaphoreType.DMA,
]


@functools.partial(
    pl.kernel,
    out_type=(
        jax.ShapeDtypeStruct((E,), jnp.float32),
        jax.ShapeDtypeStruct((E,), jnp.float32),
        jax.ShapeDtypeStruct((E,), jnp.float32),
        jax.ShapeDtypeStruct((E,), jnp.float32),
    ),
    mesh=_mesh,
    compiler_params=pltpu.CompilerParams(
        needs_layout_passes=False, use_tc_tiling_on_sc=False),
    scratch_types=_BUF + _BUF + [pltpu.VMEM((NCELL * 9,), jnp.float32)],
)
def _edge_kernel(pos4, edge3, cellflat, csx, csy, csz,
                 vx_out, vy_out, vz_out, len_out, *scratch):
    buf0 = tuple(scratch[:10])
    buf1 = tuple(scratch[10:20])
    cellv = scratch[20]
    wid = lax.axis_index("s") * NC + lax.axis_index("c")
    lanes = lax.iota(jnp.int32, 16)

    pltpu.sync_copy(cellflat, cellv)

    def base_of(c):
        return wid * PER_W + c * K

    def start_in(c, B):
        idxs_v, idxd_v, _, _, shf = B[:5]
        sem_in = B[7]
        base = base_of(c)
        rowbase = base // SUB
        pltpu.async_copy(edge3.at[0, pl.ds(rowbase, NSUB)], idxs_v, sem_in)
        pltpu.async_copy(edge3.at[1, pl.ds(rowbase, NSUB)], idxd_v, sem_in)
        for i, cs in enumerate((csx, csy, csz)):
            pltpu.async_copy(cs.at[pl.ds(base, K)], shf.at[i], sem_in)

    def wait_in(B):
        idxs_v, idxd_v, _, _, shf = B[:5]
        sem_in = B[7]
        pltpu.make_async_copy(
            edge3.at[0, pl.ds(0, NSUB)], idxs_v, sem_in).wait()
        pltpu.make_async_copy(
            edge3.at[1, pl.ds(0, NSUB)], idxd_v, sem_in).wait()
        for i, cs in enumerate((csx, csy, csz)):
            pltpu.make_async_copy(
                cs.at[pl.ds(0, K)], shf.at[i], sem_in).wait()

    def fire_g(B):
        idxs_v, idxd_v, srcr, dstr = B[:4]
        sem_g = B[8]

        def fire(j, cr):
            pltpu.async_copy(pos4.at[idxs_v.at[j]], srcr.at[j], sem_g)
            pltpu.async_copy(pos4.at[idxd_v.at[j]], dstr.at[j], sem_g)
            return cr

        lax.fori_loop(0, NSUB, fire, 0)

    def drain_g(B):
        idxs_v, idxd_v, srcr, dstr = B[:4]
        sem_g = B[8]

        def drain(j, cr):
            pltpu.make_async_copy(
                pos4.at[idxs_v.at[j]], srcr.at[j], sem_g).wait()
            pltpu.make_async_copy(
                pos4.at[idxd_v.at[j]], dstr.at[j], sem_g).wait()
            return cr

        lax.fori_loop(0, NSUB, drain, 0)

    def compute(B):
        srcr, dstr = B[2], B[3]
        shf, vecv, lenv = B[4], B[5], B[6]

        @plsc.parallel_loop(0, K // 16, 1, unroll=4)
        def group(g):
            sub = g // GPS
            subv = jnp.full((16,), sub, jnp.int32)
            rowv = (g % GPS) * 16 + lanes
            ridx = g * 16 + lanes
            sx = plsc.load_gather(srcr, [subv, rowv, _c16(0)])
            sy = plsc.load_gather(srcr, [subv, rowv, _c16(1)])
            sz = plsc.load_gather(srcr, [subv, rowv, _c16(2)])
            b = plsc.load_gather(
                srcr, [subv, rowv, _c16(3)]).astype(jnp.int32)
            dx = plsc.load_gather(dstr, [subv, rowv, _c16(0)])
            dy = plsc.load_gather(dstr, [subv, rowv, _c16(1)])
            dz = plsc.load_gather(dstr, [subv, rowv, _c16(2)])
            s0 = plsc.load_gather(shf, [_c16(0), ridx])
            s1 = plsc.load_gather(shf, [_c16(1), ridx])
            s2 = plsc.load_gather(shf, [_c16(2), ridx])
            cb = b * 9
            vx = dx - sx \
                + s0 * plsc.load_gather(cellv, [cb]) \
                + s1 * plsc.load_gather(cellv, [cb + 3]) \
                + s2 * plsc.load_gather(cellv, [cb + 6])
            vy = dy - sy \
                + s0 * plsc.load_gather(cellv, [cb + 1]) \
                + s1 * plsc.load_gather(cellv, [cb + 4]) \
                + s2 * plsc.load_gather(cellv, [cb + 7])
            vz = dz - sz \
                + s0 * plsc.load_gather(cellv, [cb + 2]) \
                + s1 * plsc.load_gather(cellv, [cb + 5]) \
                + s2 * plsc.load_gather(cellv, [cb + 8])
            nsq = vx * vx + vy * vy + vz * vz
            yi = jnp.int32(0x5F3759DF) - (plsc.bitcast(nsq, jnp.int32) >> 1)
            y = plsc.bitcast(yi, jnp.float32)
            y = y * (1.5 - 0.5 * nsq * y * y)
            y = y * (1.5 - 0.5 * nsq * y * y)
            y = y * (1.5 - 0.5 * nsq * y * y)
            ln = jnp.where(nsq > 0.0, nsq * y, 0.0)
            plsc.store_scatter(vecv, [_c16(0), ridx], vx)
            plsc.store_scatter(vecv, [_c16(1), ridx], vy)
            plsc.store_scatter(vecv, [_c16(2), ridx], vz)
            plsc.store_scatter(lenv, [ridx], ln)

    def start_out(c, B):
        vecv, lenv = B[5], B[6]
        sem_out = B[9]
        base = base_of(c)
        for i, vo in enumerate((vx_out, vy_out, vz_out)):
            pltpu.async_copy(vecv.at[i], vo.at[pl.ds(base, K)], sem_out)
        pltpu.async_copy(lenv, len_out.at[pl.ds(base, K)], sem_out)

    def wait_out(B):
        vecv, lenv = B[5], B[6]
        sem_out = B[9]
        for i, vo in enumerate((vx_out, vy_out, vz_out)):
            pltpu.make_async_copy(
                vecv.at[i], vo.at[pl.ds(0, K)], sem_out).wait()
        pltpu.make_async_copy(lenv, len_out.at[pl.ds(0, K)], sem_out).wait()

    def step(c, B, NB, do_next, do_waitout, do_startin):
        drain_g(B)
        if do_next:
            wait_in(NB)
            fire_g(NB)
        if do_waitout:
            wait_out(B)
        compute(B)
        start_out(c, B)
        if do_startin:
            start_in(c + 2, B)

    start_in(0, buf0)
    start_in(1, buf1)
    wait_in(buf0)
    fire_g(buf0)
    step(0, buf0, buf1, True, False, True)
    step(1, buf1, buf0, True, False, True)

    def body(c2, carry):
        c = 2 * c2
        step(c, buf0, buf1, True, True, True)
        step(c + 1, buf1, buf0, True, True, True)
        return carry

    lax.fori_loop(1, NCHUNK // 2 - 1, body, 0)
    step(NCHUNK - 2, buf0, buf1, True, True, False)
    step(NCHUNK - 1, buf1, buf0, False, True, False)
    wait_out(buf0)
    wait_out(buf1)


def kernel(pos, edge_index, cell, cell_shift, batch):
    pos4 = jnp.concatenate(
        [pos, batch.astype(jnp.float32)[:, None],
         jnp.zeros((pos.shape[0], 4), jnp.float32)], axis=1)
    cellflat = cell.reshape(-1)
    edge3 = edge_index.reshape(2, E // SUB, SUB)
    vx, vy, vz, length = _edge_kernel(
        pos4, edge3, cellflat,
        cell_shift[:, 0], cell_shift[:, 1], cell_shift[:, 2])
    return jnp.stack([vx, vy, vz], axis=1), length
